# baseline (speedup 1.0000x reference)
import functools
import math

import jax
import jax.numpy as jnp
from jax import lax
from jax.experimental import pallas as pl
from jax.experimental.pallas import tpu as pltpu
from jax.experimental.pallas import tpu_sc as plsc

VOCAB = 100000
EMBED_DIM = 128
BATCH = 1024
SEQ = 200

_NC = 2
_NS = 16
_NW = _NC * _NS

_TOTAL = BATCH * SEQ
_CHUNK = 64
_PER_W = _TOTAL // _NW
_NCHUNK = _PER_W // _CHUNK


_NBUF = 10
_LEAD = 5


def _sc_gather_body(table_hbm, idx_hbm, out_hbm, idx_v, bufs, semg, semp):
    wid = lax.axis_index("s") * _NC + lax.axis_index("c")
    base = wid * _PER_W
    pltpu.sync_copy(idx_hbm.at[pl.ds(base, _PER_W)], idx_v)

    def gather(g, b):
        pltpu.async_copy(
            table_hbm.at[idx_v.at[pl.ds(g * _CHUNK, _CHUNK)]], bufs.at[b], semg.at[b]
        )

    def wait_gather(g, b):
        pltpu.make_async_copy(
            table_hbm.at[idx_v.at[pl.ds(g * _CHUNK, _CHUNK)]], bufs.at[b], semg.at[b]
        ).wait()

    def put(g, b):
        pltpu.async_copy(bufs.at[b], out_hbm.at[pl.ds(base + g * _CHUNK, _CHUNK)], semp.at[b])

    def wait_put(g, b):
        pltpu.make_async_copy(
            bufs.at[b], out_hbm.at[pl.ds(base + g * _CHUNK, _CHUNK)], semp.at[b]
        ).wait()

    for g in range(_LEAD):
        gather(g, g % _NBUF)

    for b in range(_NBUF):
        g = b
        nb = (b + _LEAD) % _NBUF
        if g + _LEAD >= _NBUF:
            wait_put(g + _LEAD - _NBUF, nb)
        gather(g + _LEAD, nb)
        wait_gather(g, b)
        put(g, b)

    def steady(i, _):
        g0 = i * _NBUF
        for b in range(_NBUF):
            g = g0 + b
            nb = (b + _LEAD) % _NBUF
            wait_put(g + _LEAD - _NBUF, nb)
            gather(g + _LEAD, nb)
            wait_gather(g, b)
            put(g, b)
        return 0

    lax.fori_loop(1, _NCHUNK // _NBUF - 1, steady, 0)

    g0 = _NCHUNK - _NBUF
    for b in range(_NBUF):
        g = g0 + b
        if g + _LEAD < _NCHUNK:
            nb = (b + _LEAD) % _NBUF
            wait_put(g + _LEAD - _NBUF, nb)
            gather(g + _LEAD, nb)
        wait_gather(g, b)
        put(g, b)
    for b in range(_NBUF):
        wait_put(g0 + b, b)


@jax.jit
def _sc_gather(table, idx2d):
    mesh = plsc.VectorSubcoreMesh(core_axis_name="c", subcore_axis_name="s")
    return pl.kernel(
        _sc_gather_body,
        out_type=jax.ShapeDtypeStruct((_TOTAL, EMBED_DIM), jnp.float32),
        mesh=mesh,
        scratch_types=[
            pltpu.VMEM((_PER_W,), jnp.int32),
            pltpu.VMEM((_NBUF, _CHUNK, EMBED_DIM), jnp.float32),
            pltpu.SemaphoreType.DMA((_NBUF,)),
            pltpu.SemaphoreType.DMA((_NBUF,)),
        ],
    )(table, idx2d)


def _pe_body(out_ref):
    shape = (SEQ, EMBED_DIM)
    pos = lax.broadcasted_iota(jnp.int32, shape, 0).astype(jnp.float32)
    i = lax.broadcasted_iota(jnp.int32, shape, 1)
    two_floor = (2 * (i // 2)).astype(jnp.float32)
    inv_freq = jnp.exp(two_floor * (-math.log(10000.0) / float(EMBED_DIM)))
    angle = pos * inv_freq
    odd = (i % 2) == 1
    out_ref[...] = jnp.where(odd, jnp.cos(angle), jnp.sin(angle))


@jax.jit
def _pos_encoding():
    return pl.pallas_call(
        _pe_body,
        out_shape=jax.ShapeDtypeStruct((SEQ, EMBED_DIM), jnp.float32),
    )()


def kernel(inputs, table):
    idx_flat = inputs.reshape(_TOTAL)
    embed = _sc_gather(table, idx_flat)
    pe = _pos_encoding()
    return embed.reshape(BATCH, SEQ, EMBED_DIM), pe


# speedup vs baseline: 1.0049x; 1.0049x over previous
import functools
import math

import jax
import jax.numpy as jnp
from jax import lax
from jax.experimental import pallas as pl
from jax.experimental.pallas import tpu as pltpu
from jax.experimental.pallas import tpu_sc as plsc

VOCAB = 100000
EMBED_DIM = 128
BATCH = 1024
SEQ = 200

_NC = 2
_NS = 16
_NW = _NC * _NS

_TOTAL = BATCH * SEQ
_CHUNK = 80
_PER_W = _TOTAL // _NW
_NCHUNK = _PER_W // _CHUNK


_NBUF = 10
_LEAD = 5


def _sc_gather_body(table_hbm, idx_hbm, out_hbm, idx_v, bufs, semg, semp):
    wid = lax.axis_index("s") * _NC + lax.axis_index("c")
    base = wid * _PER_W
    pltpu.sync_copy(idx_hbm.at[pl.ds(base, _PER_W)], idx_v)

    def gather(g, b):
        pltpu.async_copy(
            table_hbm.at[idx_v.at[pl.ds(g * _CHUNK, _CHUNK)]], bufs.at[b], semg.at[b]
        )

    def wait_gather(g, b):
        pltpu.make_async_copy(
            table_hbm.at[idx_v.at[pl.ds(g * _CHUNK, _CHUNK)]], bufs.at[b], semg.at[b]
        ).wait()

    def put(g, b):
        pltpu.async_copy(bufs.at[b], out_hbm.at[pl.ds(base + g * _CHUNK, _CHUNK)], semp.at[b])

    def wait_put(g, b):
        pltpu.make_async_copy(
            bufs.at[b], out_hbm.at[pl.ds(base + g * _CHUNK, _CHUNK)], semp.at[b]
        ).wait()

    for g in range(_LEAD):
        gather(g, g % _NBUF)

    for b in range(_NBUF):
        g = b
        nb = (b + _LEAD) % _NBUF
        if g + _LEAD >= _NBUF:
            wait_put(g + _LEAD - _NBUF, nb)
        gather(g + _LEAD, nb)
        wait_gather(g, b)
        put(g, b)

    def steady(i, _):
        g0 = i * _NBUF
        for b in range(_NBUF):
            g = g0 + b
            nb = (b + _LEAD) % _NBUF
            wait_put(g + _LEAD - _NBUF, nb)
            gather(g + _LEAD, nb)
            wait_gather(g, b)
            put(g, b)
        return 0

    lax.fori_loop(1, _NCHUNK // _NBUF - 1, steady, 0)

    g0 = _NCHUNK - _NBUF
    for b in range(_NBUF):
        g = g0 + b
        if g + _LEAD < _NCHUNK:
            nb = (b + _LEAD) % _NBUF
            wait_put(g + _LEAD - _NBUF, nb)
            gather(g + _LEAD, nb)
        wait_gather(g, b)
        put(g, b)
    for b in range(_NBUF):
        wait_put(g0 + b, b)


@jax.jit
def _sc_gather(table, idx2d):
    mesh = plsc.VectorSubcoreMesh(core_axis_name="c", subcore_axis_name="s")
    return pl.kernel(
        _sc_gather_body,
        out_type=jax.ShapeDtypeStruct((_TOTAL, EMBED_DIM), jnp.float32),
        mesh=mesh,
        scratch_types=[
            pltpu.VMEM((_PER_W,), jnp.int32),
            pltpu.VMEM((_NBUF, _CHUNK, EMBED_DIM), jnp.float32),
            pltpu.SemaphoreType.DMA((_NBUF,)),
            pltpu.SemaphoreType.DMA((_NBUF,)),
        ],
    )(table, idx2d)


def _pe_body(out_ref):
    shape = (SEQ, EMBED_DIM)
    pos = lax.broadcasted_iota(jnp.int32, shape, 0).astype(jnp.float32)
    i = lax.broadcasted_iota(jnp.int32, shape, 1)
    two_floor = (2 * (i // 2)).astype(jnp.float32)
    inv_freq = jnp.exp(two_floor * (-math.log(10000.0) / float(EMBED_DIM)))
    angle = pos * inv_freq
    odd = (i % 2) == 1
    out_ref[...] = jnp.where(odd, jnp.cos(angle), jnp.sin(angle))


@jax.jit
def _pos_encoding():
    return pl.pallas_call(
        _pe_body,
        out_shape=jax.ShapeDtypeStruct((SEQ, EMBED_DIM), jnp.float32),
    )()


def kernel(inputs, table):
    idx_flat = inputs.reshape(_TOTAL)
    embed = _sc_gather(table, idx_flat)
    pe = _pos_encoding()
    return embed.reshape(BATCH, SEQ, EMBED_DIM), pe
